# baseline (speedup 1.0000x reference)
import dataclasses
import functools

import jax
import jax.numpy as jnp
from jax import lax
from jax.experimental import pallas as pl
from jax.experimental.pallas import tpu as pltpu
from jax.experimental.pallas import tpu_sc as plsc


LANES = 16


def kernel(x, x_mask, table):
    B, L = x.shape
    V, E = table.shape
    info = plsc.get_sparse_core_info()
    NC, NS = info.num_cores, info.num_subcores
    NW = NC * NS
    assert B % NW == 0
    rows_per = B // NW
    assert L % 2 == 0
    H = L // 2
    assert H <= 128
    EV = E // LANES
    assert E % LANES == 0

    L_pad = ((L + LANES - 1) // LANES) * LANES
    MCH = L_pad // LANES
    xm = jnp.pad(x_mask, ((0, 0), (0, L_pad - L)))
    xr = x.reshape(B * 2, H)

    mesh = plsc.VectorSubcoreMesh(core_axis_name="c", subcore_axis_name="s")
    cp = pltpu.CompilerParams()
    if "needs_layout_passes" in pltpu.CompilerParams.__dataclass_fields__:
        cp = dataclasses.replace(cp, needs_layout_passes=False)
    if "use_tc_tiling_on_sc" in pltpu.CompilerParams.__dataclass_fields__:
        cp = dataclasses.replace(cp, use_tc_tiling_on_sc=False)

    @functools.partial(
        pl.kernel,
        out_type=jax.ShapeDtypeStruct((B, E), jnp.float32),
        mesh=mesh,
        compiler_params=cp,
        scratch_types=[
            pltpu.VMEM((2 * rows_per, H), jnp.int32),
            pltpu.VMEM((rows_per, L_pad), jnp.float32),
            pltpu.VMEM((H, E), jnp.float32),
            pltpu.VMEM((H, E), jnp.float32),
            pltpu.VMEM((rows_per, E), jnp.float32),
            pltpu.SemaphoreType.DMA,
            pltpu.SemaphoreType.DMA,
        ],
    )
    def run(xr_hbm, xm_hbm, tbl_hbm, out_hbm,
            idx_v, mask_v, buf_a, buf_b, out_stage, sem_a, sem_b):
        wid = lax.axis_index("s") * NC + lax.axis_index("c")
        base = wid * rows_per
        pltpu.sync_copy(xr_hbm.at[pl.ds(wid * 2 * rows_per, 2 * rows_per)],
                        idx_v)
        pltpu.sync_copy(xm_hbm.at[pl.ds(base, rows_per)], mask_v)

        @pl.loop(0, rows_per)
        def _(i):
            cp_a = pltpu.async_copy(tbl_hbm.at[idx_v.at[2 * i]], buf_a, sem_a)
            cp_b = pltpu.async_copy(tbl_hbm.at[idx_v.at[2 * i + 1]], buf_b,
                                    sem_b)
            msum = jnp.zeros((LANES,), jnp.float32)
            for j in range(MCH):
                msum = msum + mask_v[i, pl.ds(j * LANES, LANES)]
            num = jnp.sum(msum)
            inv = 1.0 / (jnp.full((LANES,), num, jnp.float32) + 1e-20)

            cp_a.wait()
            cp_b.wait()

            def red(buf):
                def body(j, accs):
                    return tuple(
                        accs[k] + buf[j, pl.ds(k * LANES, LANES)]
                        for k in range(EV))
                return body

            zero = tuple(jnp.zeros((LANES,), jnp.float32) for _ in range(EV))
            accs = lax.fori_loop(0, H, red(buf_a), zero)
            accs = lax.fori_loop(0, H, red(buf_b), accs)
            for k in range(EV):
                out_stage[i, pl.ds(k * LANES, LANES)] = accs[k] * inv

        pltpu.sync_copy(out_stage, out_hbm.at[pl.ds(base, rows_per)])

    return run(xr, xm, table)


# speedup vs baseline: 1.1540x; 1.1540x over previous
import dataclasses
import functools

import jax
import jax.numpy as jnp
from jax import lax
from jax.experimental import pallas as pl
from jax.experimental.pallas import tpu as pltpu
from jax.experimental.pallas import tpu_sc as plsc


LANES = 16


def kernel(x, x_mask, table):
    B, L = x.shape
    V, E = table.shape
    info = plsc.get_sparse_core_info()
    NC, NS = info.num_cores, info.num_subcores
    NW = NC * NS
    assert B % (2 * NW) == 0
    rows_per = B // NW
    assert L % 4 == 0
    H = L // 2
    assert H <= 128
    EV = E // LANES
    assert E % LANES == 0

    L_pad = ((L + LANES - 1) // LANES) * LANES
    MCH = L_pad // LANES
    xm = jnp.pad(x_mask, ((0, 0), (0, L_pad - L)))
    xr = x.reshape(B * 2, H)

    mesh = plsc.VectorSubcoreMesh(core_axis_name="c", subcore_axis_name="s")
    cp = pltpu.CompilerParams()
    if "needs_layout_passes" in pltpu.CompilerParams.__dataclass_fields__:
        cp = dataclasses.replace(cp, needs_layout_passes=False)
    if "use_tc_tiling_on_sc" in pltpu.CompilerParams.__dataclass_fields__:
        cp = dataclasses.replace(cp, use_tc_tiling_on_sc=False)

    @functools.partial(
        pl.kernel,
        out_type=jax.ShapeDtypeStruct((B, E), jnp.float32),
        mesh=mesh,
        compiler_params=cp,
        scratch_types=[
            pltpu.VMEM((2 * rows_per, H), jnp.int32),
            pltpu.VMEM((rows_per, L_pad), jnp.float32),
            pltpu.VMEM((L, E), jnp.float32),
            pltpu.VMEM((L, E), jnp.float32),
            pltpu.VMEM((rows_per, E), jnp.float32),
            pltpu.SemaphoreType.DMA,
            pltpu.SemaphoreType.DMA,
        ],
    )
    def run(xr_hbm, xm_hbm, tbl_hbm, out_hbm,
            idx_v, mask_v, buf0, buf1, out_stage, sem0, sem1):
        wid = lax.axis_index("s") * NC + lax.axis_index("c")
        base = wid * rows_per
        pltpu.sync_copy(xr_hbm.at[pl.ds(wid * 2 * rows_per, 2 * rows_per)],
                        idx_v)
        pltpu.sync_copy(xm_hbm.at[pl.ds(base, rows_per)], mask_v)

        def issue(row, buf, sem):
            pltpu.async_copy(tbl_hbm.at[idx_v.at[2 * row]],
                             buf.at[pl.ds(0, H)], sem)
            pltpu.async_copy(tbl_hbm.at[idx_v.at[2 * row + 1]],
                             buf.at[pl.ds(H, H)], sem)

        def drain(buf, sem):
            pltpu.make_async_copy(tbl_hbm.at[pl.ds(0, L)], buf, sem).wait()

        def inv_num(row):
            msum = jnp.zeros((LANES,), jnp.float32)
            for j in range(MCH):
                msum = msum + mask_v[row, pl.ds(j * LANES, LANES)]
            num = jnp.sum(msum)
            return 1.0 / (jnp.full((LANES,), num, jnp.float32) + 1e-20)

        def reduce_store(row, buf, inv):
            def body(j, accs):
                a0, a1 = accs
                r0 = tuple(
                    a0[k] + (buf[4 * j, pl.ds(k * LANES, LANES)]
                             + buf[4 * j + 1, pl.ds(k * LANES, LANES)])
                    for k in range(EV))
                r1 = tuple(
                    a1[k] + (buf[4 * j + 2, pl.ds(k * LANES, LANES)]
                             + buf[4 * j + 3, pl.ds(k * LANES, LANES)])
                    for k in range(EV))
                return (r0, r1)

            zero = tuple(jnp.zeros((LANES,), jnp.float32) for _ in range(EV))
            a0, a1 = lax.fori_loop(0, L // 4, body, (zero, zero))
            for k in range(EV):
                out_stage[row, pl.ds(k * LANES, LANES)] = (a0[k] + a1[k]) * inv

        issue(0, buf0, sem0)

        @pl.loop(0, rows_per // 2)
        def _(t):
            r0 = 2 * t
            issue(r0 + 1, buf1, sem1)
            inv0 = inv_num(r0)
            drain(buf0, sem0)
            reduce_store(r0, buf0, inv0)

            @pl.when(r0 + 2 < rows_per)
            def _():
                issue(r0 + 2, buf0, sem0)

            inv1 = inv_num(r0 + 1)
            drain(buf1, sem1)
            reduce_store(r0 + 1, buf1, inv1)

        pltpu.sync_copy(out_stage, out_hbm.at[pl.ds(base, rows_per)])

    return run(xr, xm, table)


# speedup vs baseline: 1.1641x; 1.0088x over previous
import dataclasses
import functools

import jax
import jax.numpy as jnp
from jax import lax
from jax.experimental import pallas as pl
from jax.experimental.pallas import tpu as pltpu
from jax.experimental.pallas import tpu_sc as plsc


LANES = 16


def kernel(x, x_mask, table):
    B, L = x.shape
    V, E = table.shape
    info = plsc.get_sparse_core_info()
    NC, NS = info.num_cores, info.num_subcores
    NW = NC * NS
    assert B % (2 * NW) == 0
    rows_per = B // NW
    assert L % 4 == 0
    H0 = min(128, (L // 2 + 7) // 8 * 8)
    H1 = L - H0
    assert 0 < H1 <= 128 and H0 % 8 == 0 and H1 % 8 == 0
    EV = E // LANES
    assert E % LANES == 0
    MCH = L // LANES
    TAIL = L - MCH * LANES

    mesh = plsc.VectorSubcoreMesh(core_axis_name="c", subcore_axis_name="s")
    cp = pltpu.CompilerParams()
    if "needs_layout_passes" in pltpu.CompilerParams.__dataclass_fields__:
        cp = dataclasses.replace(cp, needs_layout_passes=False)
    if "use_tc_tiling_on_sc" in pltpu.CompilerParams.__dataclass_fields__:
        cp = dataclasses.replace(cp, use_tc_tiling_on_sc=False)

    @functools.partial(
        pl.kernel,
        out_type=jax.ShapeDtypeStruct((B, E), jnp.float32),
        mesh=mesh,
        compiler_params=cp,
        scratch_types=[
            pltpu.VMEM((rows_per, L), jnp.int32),
            pltpu.VMEM((rows_per, L), jnp.float32),
            pltpu.VMEM((L, E), jnp.float32),
            pltpu.VMEM((L, E), jnp.float32),
            pltpu.VMEM((rows_per, E), jnp.float32),
            pltpu.SemaphoreType.DMA,
            pltpu.SemaphoreType.DMA,
        ],
    )
    def run(x_hbm, xm_hbm, tbl_hbm, out_hbm,
            idx_v, mask_v, buf0, buf1, out_stage, sem0, sem1):
        wid = lax.axis_index("s") * NC + lax.axis_index("c")
        base = wid * rows_per
        pltpu.sync_copy(x_hbm.at[pl.ds(base, rows_per)], idx_v)
        pltpu.sync_copy(xm_hbm.at[pl.ds(base, rows_per)], mask_v)

        def issue(row, buf, sem):
            pltpu.async_copy(tbl_hbm.at[idx_v.at[row, pl.ds(0, H0)]],
                             buf.at[pl.ds(0, H0)], sem)
            pltpu.async_copy(tbl_hbm.at[idx_v.at[row, pl.ds(H0, H1)]],
                             buf.at[pl.ds(H0, H1)], sem)

        def drain(buf, sem):
            pltpu.make_async_copy(tbl_hbm.at[pl.ds(0, L)], buf, sem).wait()

        lane_ge_tail = lax.iota(jnp.int32, LANES) >= (LANES - TAIL)

        def inv_num(row):
            msum = jnp.zeros((LANES,), jnp.float32)
            for j in range(MCH):
                msum = msum + mask_v[row, pl.ds(j * LANES, LANES)]
            tail = mask_v[row, pl.ds(L - LANES, LANES)]
            msum = msum + jnp.where(lane_ge_tail, tail, 0.0)
            num = jnp.sum(msum)
            return 1.0 / (jnp.full((LANES,), num, jnp.float32) + 1e-20)

        def reduce_store(row, buf, inv):
            def body(j, accs):
                a0, a1 = accs
                r0 = tuple(
                    a0[k] + (buf[4 * j, pl.ds(k * LANES, LANES)]
                             + buf[4 * j + 1, pl.ds(k * LANES, LANES)])
                    for k in range(EV))
                r1 = tuple(
                    a1[k] + (buf[4 * j + 2, pl.ds(k * LANES, LANES)]
                             + buf[4 * j + 3, pl.ds(k * LANES, LANES)])
                    for k in range(EV))
                return (r0, r1)

            zero = tuple(jnp.zeros((LANES,), jnp.float32) for _ in range(EV))
            a0, a1 = lax.fori_loop(0, L // 4, body, (zero, zero))
            for k in range(EV):
                out_stage[row, pl.ds(k * LANES, LANES)] = (a0[k] + a1[k]) * inv

        issue(0, buf0, sem0)

        @pl.loop(0, rows_per // 2)
        def _(t):
            r0 = 2 * t
            issue(r0 + 1, buf1, sem1)
            inv0 = inv_num(r0)
            drain(buf0, sem0)
            reduce_store(r0, buf0, inv0)

            @pl.when(r0 + 2 < rows_per)
            def _():
                issue(r0 + 2, buf0, sem0)

            inv1 = inv_num(r0 + 1)
            drain(buf1, sem1)
            reduce_store(r0 + 1, buf1, inv1)

        pltpu.sync_copy(out_stage, out_hbm.at[pl.ds(base, rows_per)])

    return run(x, x_mask, table)
